# Initial kernel scaffold; baseline (speedup 1.0000x reference)
import jax
import jax.numpy as jnp
from jax.experimental import pallas as pl


def kernel(x, edge_index, W1, b1, W2, b2):
    raise NotImplementedError("write your pallas kernel here")


# speedup vs baseline: 14.4668x; 14.4668x over previous
import functools

import jax
import jax.numpy as jnp
from jax import lax
from jax.experimental import pallas as pl
from jax.experimental.pallas import tpu as pltpu
from jax.experimental.pallas import tpu_sc as plsc

N = 10000
E = 160000
DIN = 256
DH = 512
DOUT = 128

NC = 2
NS = 16


def _sc_mesh():
    return plsc.VectorSubcoreMesh(core_axis_name="c", subcore_axis_name="s",
                                  num_cores=NC, num_subcores=NS)


_DEG_K = 200
_EPW_DEG = E // (NC * NS)
_DW = 8


def _deg_build():
    @functools.partial(
        pl.kernel,
        out_type=(jax.ShapeDtypeStruct((N, _DW), jnp.float32),
                  jax.ShapeDtypeStruct((N, _DW), jnp.float32)),
        mesh=_sc_mesh(),
        scratch_types=[
            pltpu.VMEM((_DEG_K,), jnp.int32),
            pltpu.VMEM((_DEG_K, _DW), jnp.float32),
            pltpu.VMEM((632, _DW), jnp.float32),
            pltpu.VMEM_SHARED((N, _DW), jnp.float32),
        ],
        compiler_params=pltpu.CompilerParams(use_tc_tiling_on_sc=False),
    )
    def deg_kernel(dst_hbm, e0_hbm, out_a, out_b, dst_v, ones_v, stage_v,
                   acc):
        cid = lax.axis_index("c")
        sid = lax.axis_index("s")
        def _rows(t):
            return (t * 632, 520 if t == 15 else 632)

        pltpu.sync_copy(e0_hbm.at[pl.ds(0, _DEG_K)], ones_v)

        if True:
            for t in range(NS):
                @pl.when(sid == t)
                def _(t=t):
                    base, cnt = _rows(t)
                    pltpu.sync_copy(e0_hbm.at[pl.ds(base, cnt)],
                                    stage_v.at[pl.ds(0, cnt)])
                    pltpu.sync_copy(stage_v.at[pl.ds(0, cnt)],
                                    acc.at[pl.ds(base, cnt)])
            plsc.subcore_barrier()

            def body(b, carry):
                base = cid * (E // NC) + sid * _EPW_DEG + b * _DEG_K
                base = pl.multiple_of(base, 8)
                pltpu.sync_copy(dst_hbm.at[pl.ds(base, _DEG_K)], dst_v)
                pltpu.sync_copy(ones_v, acc.at[dst_v], add=True)
                return carry

            lax.fori_loop(0, _EPW_DEG // _DEG_K, body, 0)
            plsc.subcore_barrier()
            for i, out_hbm in enumerate((out_a, out_b)):
                @pl.when(cid == i)
                def _(out_hbm=out_hbm):
                    for t in range(NS):
                        @pl.when(sid == t)
                        def _(t=t, out_hbm=out_hbm):
                            base, cnt = _rows(t)
                            pltpu.sync_copy(acc.at[pl.ds(base, cnt)],
                                            stage_v.at[pl.ds(0, cnt)])
                            pltpu.sync_copy(stage_v.at[pl.ds(0, cnt)],
                                            out_hbm.at[pl.ds(base, cnt)])

    return deg_kernel


_deg_kernel = _deg_build()


def _prop_build(dh, k):
    epw = E // NS
    rpw = N // NS
    nb = epw // k

    @functools.partial(
        pl.kernel,
        out_type=(jax.ShapeDtypeStruct((N, dh), jnp.float32),
                  jax.ShapeDtypeStruct((N, dh), jnp.float32)),
        mesh=_sc_mesh(),
        scratch_types=[
            pltpu.VMEM((k,), jnp.int32),
            pltpu.VMEM((k,), jnp.int32),
            pltpu.VMEM((k, dh), jnp.float32),
            pltpu.VMEM_SHARED((N, dh), jnp.float32),
            pltpu.SemaphoreType.DMA,
        ],
        compiler_params=pltpu.CompilerParams(use_tc_tiling_on_sc=False),
    )
    def prop_kernel(xa, xb, src_hbm, dst_hbm, ua, ub, src_v, dst_v, rows_v,
                    acc, sem):
        cid = lax.axis_index("c")
        sid = lax.axis_index("s")

        pieces = []
        off = 0
        while off < 1000:
            c = min(k - k % 8, 1000 - off)
            pieces.append((off, c))
            off += c

        def _stage(src_at, dst_at):
            @pl.when(sid < 10)
            def _():
                for (poff, pcnt) in pieces:
                    r0 = sid * 1000 + poff
                    pltpu.sync_copy(src_at(r0, pcnt),
                                    rows_v.at[pl.ds(0, pcnt)])
                    pltpu.sync_copy(rows_v.at[pl.ds(0, pcnt)],
                                    dst_at(r0, pcnt))

        if True:
            for i, (xref, uref) in enumerate(((xa, ua), (xb, ub))):
                @pl.when(cid == i)
                def _(xref=xref, uref=uref):
                    _stage(lambda r, c: xref.at[pl.ds(r, c)],
                           lambda r, c: acc.at[pl.ds(r, c)])
                    plsc.subcore_barrier()

                    def body(b, carry):
                        base = sid * epw + b * k
                        base = pl.multiple_of(base, 8)
                        pltpu.sync_copy(src_hbm.at[pl.ds(base, k)], src_v)
                        pltpu.sync_copy(dst_hbm.at[pl.ds(base, k)], dst_v)
                        pltpu.async_copy(xref.at[src_v], rows_v, sem).wait()
                        pltpu.sync_copy(rows_v, acc.at[dst_v], add=True)
                        return carry

                    lax.fori_loop(0, nb, body, 0)
                    plsc.subcore_barrier()
                    _stage(lambda r, c: acc.at[pl.ds(r, c)],
                           lambda r, c: uref.at[pl.ds(r, c)])

    return prop_kernel


_prop64 = _prop_build(64, 400)


_RB = 1000


def _scale_body(dega_ref, degb_ref, x_ref, s_out, *xq_out):
    s = lax.rsqrt(dega_ref[:, 0:1] + degb_ref[:, 0:1] - 1.0)
    s_out[...] = s
    for q, out in enumerate(xq_out):
        out[...] = x_ref[:, q * 64:(q + 1) * 64] * s


def _scale_call(dega, degb, x):
    nq = DIN // 64
    return pl.pallas_call(
        _scale_body,
        grid=(N // _RB,),
        in_specs=[
            pl.BlockSpec((_RB, _DW), lambda i: (i, 0)),
            pl.BlockSpec((_RB, _DW), lambda i: (i, 0)),
            pl.BlockSpec((_RB, DIN), lambda i: (i, 0)),
        ],
        out_specs=[pl.BlockSpec((_RB, 1), lambda i: (i, 0))]
        + [pl.BlockSpec((_RB, 64), lambda i: (i, 0)) for _ in range(nq)],
        out_shape=[jax.ShapeDtypeStruct((N, 1), jnp.float32)]
        + [jax.ShapeDtypeStruct((N, 64), jnp.float32) for _ in range(nq)],
    )(dega, degb, x)


def _mlp_body(u0, u1, u2, u3, s_ref, w1_ref, b1_ref, w2_ref, xa_out, xb_out):
    s = s_ref[...]
    y = jnp.concatenate([u0[...], u1[...], u2[...], u3[...]], axis=1) * s
    z = jnp.dot(y, w1_ref[...], preferred_element_type=jnp.float32)
    z = jnp.maximum(z + b1_ref[...], 0.0)
    h2 = jnp.dot(z, w2_ref[...], preferred_element_type=jnp.float32)
    x2 = h2 * s
    xa_out[...] = x2[:, : DOUT // 2]
    xb_out[...] = x2[:, DOUT // 2:]


def _mlp_call(uq, s, w1, b1, w2):
    return pl.pallas_call(
        _mlp_body,
        grid=(N // _RB,),
        in_specs=[pl.BlockSpec((_RB, 64), lambda i: (i, 0))
                  for _ in range(4)]
        + [
            pl.BlockSpec((_RB, 1), lambda i: (i, 0)),
            pl.BlockSpec((DIN, DH), lambda i: (0, 0)),
            pl.BlockSpec((1, DH), lambda i: (0, 0)),
            pl.BlockSpec((DH, DOUT), lambda i: (0, 0)),
        ],
        out_specs=[
            pl.BlockSpec((_RB, DOUT // 2), lambda i: (i, 0)),
            pl.BlockSpec((_RB, DOUT // 2), lambda i: (i, 0)),
        ],
        out_shape=[
            jax.ShapeDtypeStruct((N, DOUT // 2), jnp.float32),
            jax.ShapeDtypeStruct((N, DOUT // 2), jnp.float32),
        ],
    )(*uq, s, w1, b1.reshape(1, DH), w2)


def _final_body(ua_ref, ub_ref, s_ref, b2_ref, out_ref):
    u = jnp.concatenate([ua_ref[...], ub_ref[...]], axis=1)
    out_ref[...] = u * s_ref[...] + b2_ref[...]


def _final_call(ua, ub, s, b2):
    return pl.pallas_call(
        _final_body,
        grid=(N // _RB,),
        in_specs=[
            pl.BlockSpec((_RB, DOUT // 2), lambda i: (i, 0)),
            pl.BlockSpec((_RB, DOUT // 2), lambda i: (i, 0)),
            pl.BlockSpec((_RB, 1), lambda i: (i, 0)),
            pl.BlockSpec((1, DOUT), lambda i: (0, 0)),
        ],
        out_specs=pl.BlockSpec((_RB, DOUT), lambda i: (i, 0)),
        out_shape=jax.ShapeDtypeStruct((N, DOUT), jnp.float32),
    )(ua, ub, s, b2.reshape(1, DOUT))


def kernel(x, edge_index, W1, b1, W2, b2):
    src = edge_index[0]
    dst = edge_index[1]
    e0 = jnp.zeros((N, _DW), jnp.float32).at[:, 0].set(1.0)

    dega, degb = _deg_kernel(dst, e0)
    s, x1q0, x1q1, x1q2, x1q3 = _scale_call(dega, degb, x)
    u1q0, u1q1 = _prop64(x1q0, x1q1, src, dst)
    u1q2, u1q3 = _prop64(x1q2, x1q3, src, dst)
    x2a, x2b = _mlp_call((u1q0, u1q1, u1q2, u1q3), s, W1, b1, W2)
    u2a, u2b = _prop64(x2a, x2b, src, dst)
    return _final_call(u2a, u2b, s, b2)


# speedup vs baseline: 20.5666x; 1.4216x over previous
import functools

import jax
import jax.numpy as jnp
from jax import lax
from jax.experimental import pallas as pl
from jax.experimental.pallas import tpu as pltpu
from jax.experimental.pallas import tpu_sc as plsc

N = 10000
E = 160000
DIN = 256
DH = 512
DOUT = 128

NC = 2
NS = 16


def _sc_mesh():
    return plsc.VectorSubcoreMesh(core_axis_name="c", subcore_axis_name="s",
                                  num_cores=NC, num_subcores=NS)


_DEG_K = 200
_EPW_DEG = E // (NC * NS)
_DW = 8


def _deg_build():
    @functools.partial(
        pl.kernel,
        out_type=(jax.ShapeDtypeStruct((N, _DW), jnp.float32),
                  jax.ShapeDtypeStruct((N, _DW), jnp.float32)),
        mesh=_sc_mesh(),
        scratch_types=[
            pltpu.VMEM((_DEG_K,), jnp.int32),
            pltpu.VMEM((_DEG_K, _DW), jnp.float32),
            pltpu.VMEM((632, _DW), jnp.float32),
            pltpu.VMEM_SHARED((N, _DW), jnp.float32),
        ],
        compiler_params=pltpu.CompilerParams(use_tc_tiling_on_sc=False),
    )
    def deg_kernel(dst_hbm, e0_hbm, out_a, out_b, dst_v, ones_v, stage_v,
                   acc):
        cid = lax.axis_index("c")
        sid = lax.axis_index("s")
        def _rows(t):
            return (t * 632, 520 if t == 15 else 632)

        pltpu.sync_copy(e0_hbm.at[pl.ds(0, _DEG_K)], ones_v)

        if True:
            for t in range(NS):
                @pl.when(sid == t)
                def _(t=t):
                    base, cnt = _rows(t)
                    pltpu.sync_copy(e0_hbm.at[pl.ds(base, cnt)],
                                    stage_v.at[pl.ds(0, cnt)])
                    pltpu.sync_copy(stage_v.at[pl.ds(0, cnt)],
                                    acc.at[pl.ds(base, cnt)])
            plsc.subcore_barrier()

            def body(b, carry):
                base = cid * (E // NC) + sid * _EPW_DEG + b * _DEG_K
                base = pl.multiple_of(base, 8)
                pltpu.sync_copy(dst_hbm.at[pl.ds(base, _DEG_K)], dst_v)
                pltpu.sync_copy(ones_v, acc.at[dst_v], add=True)
                return carry

            lax.fori_loop(0, _EPW_DEG // _DEG_K, body, 0)
            plsc.subcore_barrier()
            for i, out_hbm in enumerate((out_a, out_b)):
                @pl.when(cid == i)
                def _(out_hbm=out_hbm):
                    for t in range(NS):
                        @pl.when(sid == t)
                        def _(t=t, out_hbm=out_hbm):
                            base, cnt = _rows(t)
                            pltpu.sync_copy(acc.at[pl.ds(base, cnt)],
                                            stage_v.at[pl.ds(0, cnt)])
                            pltpu.sync_copy(stage_v.at[pl.ds(0, cnt)],
                                            out_hbm.at[pl.ds(base, cnt)])

    return deg_kernel


_deg_kernel = _deg_build()


def _prop_build(dh, k):
    epw = E // NS
    nb = epw // k

    @functools.partial(
        pl.kernel,
        out_type=(jax.ShapeDtypeStruct((N, dh), jnp.float32),
                  jax.ShapeDtypeStruct((N, dh), jnp.float32)),
        mesh=_sc_mesh(),
        scratch_types=[
            pltpu.VMEM((nb, k), jnp.int32),
            pltpu.VMEM((nb, k), jnp.int32),
            pltpu.VMEM((k, dh), jnp.float32),
            pltpu.VMEM((k, dh), jnp.float32),
            pltpu.VMEM_SHARED((N, dh), jnp.float32),
            pltpu.SemaphoreType.DMA,
            pltpu.SemaphoreType.DMA,
        ],
        compiler_params=pltpu.CompilerParams(use_tc_tiling_on_sc=False),
    )
    def prop_kernel(xa, xb, src_hbm, dst_hbm, ua, ub, src_v, dst_v, rows0,
                    rows1, acc, sem0, sem1):
        cid = lax.axis_index("c")
        sid = lax.axis_index("s")

        pieces = []
        off = 0
        while off < 1000:
            c = min(k - k % 8, 1000 - off)
            pieces.append((off, c))
            off += c

        def _stage(src_at, dst_at):
            @pl.when(sid < 10)
            def _():
                for (poff, pcnt) in pieces:
                    r0 = sid * 1000 + poff
                    pltpu.sync_copy(src_at(r0, pcnt),
                                    rows0.at[pl.ds(0, pcnt)])
                    pltpu.sync_copy(rows0.at[pl.ds(0, pcnt)],
                                    dst_at(r0, pcnt))

        pltpu.sync_copy(src_hbm.at[sid], src_v)
        pltpu.sync_copy(dst_hbm.at[sid], dst_v)

        for i, (xref, uref) in enumerate(((xa, ua), (xb, ub))):
            @pl.when(cid == i)
            def _(xref=xref, uref=uref):
                _stage(lambda r, c: xref.at[pl.ds(r, c)],
                       lambda r, c: acc.at[pl.ds(r, c)])
                plsc.subcore_barrier()

                pltpu.async_copy(xref.at[src_v.at[0]], rows0, sem0)

                def pair(i2, carry):
                    b1 = 2 * i2 + 1
                    b2 = 2 * i2 + 2

                    @pl.when(b1 < nb)
                    def _():
                        pltpu.async_copy(xref.at[src_v.at[b1]], rows1, sem1)
                    pltpu.make_async_copy(
                        xref.at[src_v.at[0]], rows0, sem0).wait()
                    pltpu.sync_copy(rows0, acc.at[dst_v.at[2 * i2]],
                                    add=True)

                    @pl.when(b2 < nb)
                    def _():
                        pltpu.async_copy(xref.at[src_v.at[b2]], rows0, sem0)

                    @pl.when(b1 < nb)
                    def _():
                        pltpu.make_async_copy(
                            xref.at[src_v.at[0]], rows1, sem1).wait()
                        pltpu.sync_copy(rows1, acc.at[dst_v.at[b1]],
                                        add=True)
                    return carry

                lax.fori_loop(0, (nb + 1) // 2, pair, 0)
                plsc.subcore_barrier()
                _stage(lambda r, c: acc.at[pl.ds(r, c)],
                       lambda r, c: uref.at[pl.ds(r, c)])

    return prop_kernel


_prop64 = _prop_build(64, 400)


_RB = 1000


def _scale_body(dega_ref, degb_ref, x_ref, s_out, *xq_out):
    s = lax.rsqrt(dega_ref[:, 0:1] + degb_ref[:, 0:1] - 1.0)
    s_out[...] = s
    for q, out in enumerate(xq_out):
        out[...] = x_ref[:, q * 64:(q + 1) * 64] * s


def _scale_call(dega, degb, x):
    nq = DIN // 64
    return pl.pallas_call(
        _scale_body,
        grid=(N // _RB,),
        in_specs=[
            pl.BlockSpec((_RB, _DW), lambda i: (i, 0)),
            pl.BlockSpec((_RB, _DW), lambda i: (i, 0)),
            pl.BlockSpec((_RB, DIN), lambda i: (i, 0)),
        ],
        out_specs=[pl.BlockSpec((_RB, 1), lambda i: (i, 0))]
        + [pl.BlockSpec((_RB, 64), lambda i: (i, 0)) for _ in range(nq)],
        out_shape=[jax.ShapeDtypeStruct((N, 1), jnp.float32)]
        + [jax.ShapeDtypeStruct((N, 64), jnp.float32) for _ in range(nq)],
    )(dega, degb, x)


def _mlp_body(u0, u1, u2, u3, s_ref, w1_ref, b1_ref, w2_ref, xa_out, xb_out):
    s = s_ref[...]
    y = jnp.concatenate([u0[...], u1[...], u2[...], u3[...]], axis=1) * s
    z = jnp.dot(y, w1_ref[...], preferred_element_type=jnp.float32)
    z = jnp.maximum(z + b1_ref[...], 0.0)
    h2 = jnp.dot(z, w2_ref[...], preferred_element_type=jnp.float32)
    x2 = h2 * s
    xa_out[...] = x2[:, : DOUT // 2]
    xb_out[...] = x2[:, DOUT // 2:]


def _mlp_call(uq, s, w1, b1, w2):
    return pl.pallas_call(
        _mlp_body,
        grid=(N // _RB,),
        in_specs=[pl.BlockSpec((_RB, 64), lambda i: (i, 0))
                  for _ in range(4)]
        + [
            pl.BlockSpec((_RB, 1), lambda i: (i, 0)),
            pl.BlockSpec((DIN, DH), lambda i: (0, 0)),
            pl.BlockSpec((1, DH), lambda i: (0, 0)),
            pl.BlockSpec((DH, DOUT), lambda i: (0, 0)),
        ],
        out_specs=[
            pl.BlockSpec((_RB, DOUT // 2), lambda i: (i, 0)),
            pl.BlockSpec((_RB, DOUT // 2), lambda i: (i, 0)),
        ],
        out_shape=[
            jax.ShapeDtypeStruct((N, DOUT // 2), jnp.float32),
            jax.ShapeDtypeStruct((N, DOUT // 2), jnp.float32),
        ],
    )(*uq, s, w1, b1.reshape(1, DH), w2)


def _final_body(ua_ref, ub_ref, s_ref, b2_ref, out_ref):
    u = jnp.concatenate([ua_ref[...], ub_ref[...]], axis=1)
    out_ref[...] = u * s_ref[...] + b2_ref[...]


def _final_call(ua, ub, s, b2):
    return pl.pallas_call(
        _final_body,
        grid=(N // _RB,),
        in_specs=[
            pl.BlockSpec((_RB, DOUT // 2), lambda i: (i, 0)),
            pl.BlockSpec((_RB, DOUT // 2), lambda i: (i, 0)),
            pl.BlockSpec((_RB, 1), lambda i: (i, 0)),
            pl.BlockSpec((1, DOUT), lambda i: (0, 0)),
        ],
        out_specs=pl.BlockSpec((_RB, DOUT), lambda i: (i, 0)),
        out_shape=jax.ShapeDtypeStruct((N, DOUT), jnp.float32),
    )(ua, ub, s, b2.reshape(1, DOUT))


def kernel(x, edge_index, W1, b1, W2, b2):
    src = edge_index[0]
    dst = edge_index[1]
    e0 = jnp.zeros((N, _DW), jnp.float32).at[:, 0].set(1.0)

    src3 = src.reshape(NS, -1, 400)
    dst3 = dst.reshape(NS, -1, 400)
    dega, degb = _deg_kernel(dst, e0)
    s, x1q0, x1q1, x1q2, x1q3 = _scale_call(dega, degb, x)
    u1q0, u1q1 = _prop64(x1q0, x1q1, src3, dst3)
    u1q2, u1q3 = _prop64(x1q2, x1q3, src3, dst3)
    x2a, x2b = _mlp_call((u1q0, u1q1, u1q2, u1q3), s, W1, b1, W2)
    u2a, u2b = _prop64(x2a, x2b, src3, dst3)
    return _final_call(u2a, u2b, s, b2)
